# Initial kernel scaffold; baseline (speedup 1.0000x reference)
import functools

import jax
import jax.numpy as jnp
from jax import lax
from jax.experimental import pallas as pl
from jax.experimental.pallas import tpu as pltpu
from jax.experimental.pallas import tpu_sc as plsc

N = 100000
D = 128
B = 16384
NC = 2
NS = 16
L = 16

NCHUNK = 10
R = N // NCHUNK
ROWS_PER_TILE = R // NS
SCAN_PER_TILE = B // NS
NVEC = SCAN_PER_TILE // L
KB = 128
MAXM = SCAN_PER_TILE + KB
NBROWS = MAXM // KB


def _sc_index_add(dst, src, idx):
  mesh = plsc.VectorSubcoreMesh(
      core_axis_name="c", subcore_axis_name="s",
      num_cores=NC, num_subcores=NS)

  @functools.partial(
      pl.kernel,
      out_type=jax.ShapeDtypeStruct((N, D), jnp.float32),
      mesh=mesh,
      scratch_types=[
          pltpu.VMEM_SHARED((R + L, D), jnp.float32),
          pltpu.VMEM((SCAN_PER_TILE,), jnp.int32),
          pltpu.VMEM((MAXM,), jnp.int32),
          pltpu.VMEM((MAXM,), jnp.int32),
          pltpu.VMEM((NBROWS, KB), jnp.int32),
          pltpu.VMEM((KB, D), jnp.float32),
          pltpu.SemaphoreType.DMA,
      ],
  )
  def run(dst_hbm, src_hbm, idx_hbm, out_hbm,
          acc, idxbuf, posbuf, lidx1d, lidx2d, staging, sem):
    core = lax.axis_index("c")
    sub = lax.axis_index("s")
    lanes = lax.iota(jnp.int32, L)

    pltpu.sync_copy(idx_hbm.at[pl.ds(sub * SCAN_PER_TILE, SCAN_PER_TILE)],
                    idxbuf)

    @pl.loop(0, NCHUNK // NC)
    def _chunk(k):
      c = k * NC + core
      base = c * R

      pltpu.async_copy(
          dst_hbm.at[pl.ds(base + sub * ROWS_PER_TILE, ROWS_PER_TILE)],
          acc.at[pl.ds(sub * ROWS_PER_TILE, ROWS_PER_TILE)], sem).wait()
      plsc.subcore_barrier()

      def scan_body(v, m):
        vec = idxbuf[pl.ds(v * L, L)]
        mask = (vec >= base) & (vec < base + R)
        mi = mask.astype(jnp.int32)
        off = m + plsc.cumsum(mi) - mi
        pos = lanes + (sub * SCAN_PER_TILE + v * L)
        plsc.store_scatter(posbuf, [off], pos, mask=mask)
        plsc.store_scatter(lidx1d, [off], vec - base, mask=mask)
        return m + jnp.max(plsc.all_reduce_population_count(mask))

      m = lax.fori_loop(0, NVEC, scan_body, jnp.int32(0))

      @pl.loop(0, KB // L)
      def _pad(j):
        posbuf[pl.ds(m + j * L, L)] = lanes + j * L
        lidx1d[pl.ds(m + j * L, L)] = lanes + R

      for r in range(NBROWS):
        for j in range(KB // L):
          lidx2d[r, pl.ds(j * L, L)] = lidx1d[pl.ds(r * KB + j * L, L)]

      nb = (m + (KB - 1)) // KB

      def batch_body(b, carry):
        pltpu.sync_copy(src_hbm.at[posbuf.at[pl.ds(b * KB, KB)]], staging)
        pltpu.sync_copy(staging, acc.at[lidx2d.at[b]], add=True)
        return carry

      lax.fori_loop(0, nb, batch_body, jnp.int32(0))
      plsc.subcore_barrier()

      pltpu.sync_copy(
          acc.at[pl.ds(sub * ROWS_PER_TILE, ROWS_PER_TILE)],
          out_hbm.at[pl.ds(base + sub * ROWS_PER_TILE, ROWS_PER_TILE)])
      plsc.subcore_barrier()

  return run(dst, src, idx)


def kernel(dst_tensor, src_tensor, index_tensor):
  return _sc_index_add(dst_tensor, src_tensor,
                       index_tensor.astype(jnp.int32))


# speedup vs baseline: 1.4773x; 1.4773x over previous
import dataclasses
import functools

import jax
import jax.numpy as jnp
from jax import lax
from jax.experimental import pallas as pl
from jax.experimental.pallas import tpu as pltpu
from jax.experimental.pallas import tpu_sc as plsc

N = 100000
D = 128
B = 16384
NC = 2
NS = 16
L = 16

NCHUNK = 10
R = N // NCHUNK
DENSE_TILES = 10
ROWS_PER_TILE = R // DENSE_TILES
SCAN_PER_TILE = B // NS
NVEC = SCAN_PER_TILE // L
KB = 128
MAXM = SCAN_PER_TILE + KB
NBROWS = MAXM // KB


def _sc_index_add(dst, src, idx):
  mesh = plsc.VectorSubcoreMesh(
      core_axis_name="c", subcore_axis_name="s",
      num_cores=NC, num_subcores=NS)
  cp = pltpu.CompilerParams()
  if "needs_layout_passes" in pltpu.CompilerParams.__dataclass_fields__:
    cp = dataclasses.replace(cp, needs_layout_passes=False)

  @functools.partial(
      pl.kernel,
      out_type=jax.ShapeDtypeStruct((N, D), jnp.float32),
      mesh=mesh,
      compiler_params=cp,
      scratch_types=[
          pltpu.VMEM_SHARED((R + L, D), jnp.float32),
          pltpu.VMEM((SCAN_PER_TILE,), jnp.int32),
          pltpu.VMEM((MAXM,), jnp.int32),
          pltpu.VMEM((MAXM,), jnp.int32),
          pltpu.VMEM((NBROWS, KB), jnp.int32),
          pltpu.VMEM((KB, D), jnp.float32),
          pltpu.SemaphoreType.DMA,
      ],
  )
  def run(dst_hbm, src_hbm, idx_hbm, out_hbm,
          acc, idxbuf, posbuf, lidx1d, lidx2d, staging, sem):
    core = lax.axis_index("c")
    sub = lax.axis_index("s")
    lanes = lax.iota(jnp.int32, L)

    pltpu.sync_copy(idx_hbm.at[pl.ds(sub * SCAN_PER_TILE, SCAN_PER_TILE)],
                    idxbuf)

    @pl.loop(0, NCHUNK // NC)
    def _chunk(k):
      c = k * NC + core
      base = c * R

      @pl.when(sub < DENSE_TILES)
      def _load():
        pltpu.async_copy(
            dst_hbm.at[pl.ds(base + sub * ROWS_PER_TILE, ROWS_PER_TILE)],
            acc.at[pl.ds(sub * ROWS_PER_TILE, ROWS_PER_TILE)], sem).wait()
      plsc.subcore_barrier()

      def scan_body(v, m):
        vec = idxbuf[pl.ds(v * L, L)]
        mask = (vec >= base) & (vec < base + R)
        mi = mask.astype(jnp.int32)
        off = m + plsc.cumsum(mi) - mi
        pos = lanes + (sub * SCAN_PER_TILE + v * L)
        plsc.store_scatter(posbuf, [off], pos, mask=mask)
        plsc.store_scatter(lidx1d, [off], vec - base, mask=mask)
        return m + jnp.max(plsc.all_reduce_population_count(mask))

      m = lax.fori_loop(0, NVEC, scan_body, jnp.int32(0))

      @pl.loop(0, KB // L)
      def _pad(j):
        posbuf[pl.ds(m + j * L, L)] = lanes + j * L
        lidx1d[pl.ds(m + j * L, L)] = lanes + R

      for r in range(NBROWS):
        for j in range(KB // L):
          lidx2d[r, pl.ds(j * L, L)] = lidx1d[pl.ds(r * KB + j * L, L)]

      nb = (m + (KB - 1)) // KB

      def batch_body(b, carry):
        pltpu.sync_copy(src_hbm.at[posbuf.at[pl.ds(b * KB, KB)]], staging)
        pltpu.sync_copy(staging, acc.at[lidx2d.at[b]], add=True)
        return carry

      lax.fori_loop(0, nb, batch_body, jnp.int32(0))
      plsc.subcore_barrier()

      @pl.when(sub < DENSE_TILES)
      def _store():
        pltpu.sync_copy(
            acc.at[pl.ds(sub * ROWS_PER_TILE, ROWS_PER_TILE)],
            out_hbm.at[pl.ds(base + sub * ROWS_PER_TILE, ROWS_PER_TILE)])
      plsc.subcore_barrier()

  return run(dst, src, idx)


def kernel(dst_tensor, src_tensor, index_tensor):
  return _sc_index_add(dst_tensor, src_tensor,
                       index_tensor.astype(jnp.int32))


# speedup vs baseline: 1.6022x; 1.0845x over previous
import dataclasses
import functools

import jax
import jax.numpy as jnp
from jax import lax
from jax.experimental import pallas as pl
from jax.experimental.pallas import tpu as pltpu
from jax.experimental.pallas import tpu_sc as plsc

N = 100000
D = 128
B = 16384
NC = 2
NS = 16
L = 16

NCHUNK = 20
R = N // NCHUNK
KPC = NCHUNK // NC
DENSE_TILES = 5
ROWS_PER_TILE = R // DENSE_TILES
SCAN_PER_TILE = B // NS
NVEC = SCAN_PER_TILE // L
KB = 128
MAXM = SCAN_PER_TILE + KB
NBROWS = MAXM // KB


def _sc_index_add(dst, src, idx):
  mesh = plsc.VectorSubcoreMesh(
      core_axis_name="c", subcore_axis_name="s",
      num_cores=NC, num_subcores=NS)
  cp = pltpu.CompilerParams()
  if "needs_layout_passes" in pltpu.CompilerParams.__dataclass_fields__:
    cp = dataclasses.replace(cp, needs_layout_passes=False)

  @functools.partial(
      pl.kernel,
      out_type=jax.ShapeDtypeStruct((N, D), jnp.float32),
      mesh=mesh,
      compiler_params=cp,
      scratch_types=[
          pltpu.VMEM_SHARED((R + L, D), jnp.float32),
          pltpu.VMEM_SHARED((R + L, D), jnp.float32),
          pltpu.VMEM((SCAN_PER_TILE,), jnp.int32),
          pltpu.VMEM((MAXM,), jnp.int32),
          pltpu.VMEM((MAXM,), jnp.int32),
          pltpu.VMEM((NBROWS, KB), jnp.int32),
          pltpu.VMEM((KB, D), jnp.float32),
          pltpu.SemaphoreType.DMA,
          pltpu.SemaphoreType.DMA,
          pltpu.SemaphoreType.DMA,
          pltpu.SemaphoreType.DMA,
      ],
  )
  def run(dst_hbm, src_hbm, idx_hbm, out_hbm,
          acc0, acc1, idxbuf, posbuf, lidx1d, lidx2d, staging,
          lsem0, lsem1, ssem0, ssem1):
    core = lax.axis_index("c")
    sub = lax.axis_index("s")
    lanes = lax.iota(jnp.int32, L)

    def hbm_slc(k_local):
      base = (k_local * NC + core) * R
      return dst_hbm.at[pl.ds(base + sub * ROWS_PER_TILE, ROWS_PER_TILE)]

    def out_slc(k_local):
      base = (k_local * NC + core) * R
      return out_hbm.at[pl.ds(base + sub * ROWS_PER_TILE, ROWS_PER_TILE)]

    def acc_slc(acc):
      return acc.at[pl.ds(sub * ROWS_PER_TILE, ROWS_PER_TILE)]

    def load_issue(k_local, acc, sem):
      pltpu.async_copy(hbm_slc(k_local), acc_slc(acc), sem)

    def load_wait(k_local, acc, sem):
      pltpu.make_async_copy(hbm_slc(k_local), acc_slc(acc), sem).wait()

    def store_issue(k_local, acc, sem):
      pltpu.async_copy(acc_slc(acc), out_slc(k_local), sem)

    def store_wait(k_local, acc, sem):
      pltpu.make_async_copy(acc_slc(acc), out_slc(k_local), sem).wait()

    def work(k_local, acc):
      base = (k_local * NC + core) * R

      def scan_body(v, m):
        vec = idxbuf[pl.ds(v * L, L)]
        rel = vec - base
        mask = rel.astype(jnp.uint32) < jnp.uint32(R)
        mi = mask.astype(jnp.int32)
        off = m + plsc.cumsum(mi) - mi
        pos = lanes + (sub * SCAN_PER_TILE + v * L)
        plsc.store_scatter(posbuf, [off], pos, mask=mask)
        plsc.store_scatter(lidx1d, [off], rel, mask=mask)
        return m + jnp.max(plsc.all_reduce_population_count(mask))

      m = lax.fori_loop(0, NVEC, scan_body, jnp.int32(0), unroll=4)

      @pl.loop(0, KB // L)
      def _pad(j):
        posbuf[pl.ds(m + j * L, L)] = lanes + j * L
        lidx1d[pl.ds(m + j * L, L)] = lanes + R

      for r in range(NBROWS):
        for j in range(KB // L):
          lidx2d[r, pl.ds(j * L, L)] = lidx1d[pl.ds(r * KB + j * L, L)]

      nb = (m + (KB - 1)) // KB

      def batch_body(b, carry):
        pltpu.sync_copy(src_hbm.at[posbuf.at[pl.ds(b * KB, KB)]], staging)
        pltpu.sync_copy(staging, acc.at[lidx2d.at[b]], add=True)
        return carry

      lax.fori_loop(0, nb, batch_body, jnp.int32(0))

    pltpu.sync_copy(idx_hbm.at[pl.ds(sub * SCAN_PER_TILE, SCAN_PER_TILE)],
                    idxbuf)

    @pl.when(sub < DENSE_TILES)
    def _prime():
      load_issue(0, acc0, lsem0)
      load_issue(1, acc1, lsem1)

    @pl.loop(0, KPC // 2)
    def _pair(j):
      k0 = 2 * j
      k1 = 2 * j + 1

      @pl.when(sub < DENSE_TILES)
      def _w0():
        load_wait(k0, acc0, lsem0)
      plsc.subcore_barrier()
      work(k0, acc0)
      plsc.subcore_barrier()

      @pl.when(sub < DENSE_TILES)
      def _s0():
        store_issue(k0, acc0, ssem0)

      @pl.when(sub < DENSE_TILES)
      def _w1():
        load_wait(k1, acc1, lsem1)
      plsc.subcore_barrier()
      work(k1, acc1)
      plsc.subcore_barrier()

      @pl.when(sub < DENSE_TILES)
      def _s1():
        store_issue(k1, acc1, ssem1)

      @pl.when(jnp.logical_and(sub < DENSE_TILES, j < KPC // 2 - 1))
      def _reissue():
        store_wait(k0, acc0, ssem0)
        load_issue(k0 + 2, acc0, lsem0)
        store_wait(k1, acc1, ssem1)
        load_issue(k1 + 2, acc1, lsem1)

    @pl.when(sub < DENSE_TILES)
    def _drain():
      store_wait(KPC - 2, acc0, ssem0)
      store_wait(KPC - 1, acc1, ssem1)

  return run(dst, src, idx)


def kernel(dst_tensor, src_tensor, index_tensor):
  return _sc_index_add(dst_tensor, src_tensor,
                       index_tensor.astype(jnp.int32))


# speedup vs baseline: 1.6269x; 1.0154x over previous
import dataclasses
import functools

import jax
import jax.numpy as jnp
from jax import lax
from jax.experimental import pallas as pl
from jax.experimental.pallas import tpu as pltpu
from jax.experimental.pallas import tpu_sc as plsc

N = 100000
D = 128
B = 16384
NC = 2
NS = 16
L = 16

NCHUNK = 20
R = N // NCHUNK
KPC = NCHUNK // NC
DENSE_TILES = 5
ROWS_PER_TILE = R // DENSE_TILES
SCAN_PER_TILE = B // NS
NVEC = SCAN_PER_TILE // L
KB = 128
MAXM = SCAN_PER_TILE + KB
NBROWS = MAXM // KB


def _sc_index_add(dst, src, idx):
  mesh = plsc.VectorSubcoreMesh(
      core_axis_name="c", subcore_axis_name="s",
      num_cores=NC, num_subcores=NS)
  cp = pltpu.CompilerParams()
  if "needs_layout_passes" in pltpu.CompilerParams.__dataclass_fields__:
    cp = dataclasses.replace(cp, needs_layout_passes=False)

  @functools.partial(
      pl.kernel,
      out_type=jax.ShapeDtypeStruct((N, D), jnp.float32),
      mesh=mesh,
      compiler_params=cp,
      scratch_types=[
          pltpu.VMEM_SHARED((R + L, D), jnp.float32),
          pltpu.VMEM_SHARED((R + L, D), jnp.float32),
          pltpu.VMEM((SCAN_PER_TILE,), jnp.int32),
          pltpu.VMEM((MAXM,), jnp.int32),
          pltpu.VMEM((NBROWS, KB), jnp.int32),
          pltpu.VMEM((KB, D), jnp.float32),
          pltpu.SemaphoreType.DMA,
          pltpu.SemaphoreType.DMA,
          pltpu.SemaphoreType.DMA,
          pltpu.SemaphoreType.DMA,
      ],
  )
  def run(dst_hbm, src_hbm, idx_hbm, out_hbm,
          acc0, acc1, idxbuf, posbuf, lidx2d, staging,
          lsem0, lsem1, ssem0, ssem1):
    core = lax.axis_index("c")
    sub = lax.axis_index("s")
    lanes = lax.iota(jnp.int32, L)

    def hbm_slc(k_local):
      base = (k_local * NC + core) * R
      return dst_hbm.at[pl.ds(base + sub * ROWS_PER_TILE, ROWS_PER_TILE)]

    def out_slc(k_local):
      base = (k_local * NC + core) * R
      return out_hbm.at[pl.ds(base + sub * ROWS_PER_TILE, ROWS_PER_TILE)]

    def acc_slc(acc):
      return acc.at[pl.ds(sub * ROWS_PER_TILE, ROWS_PER_TILE)]

    def load_issue(k_local, acc, sem):
      pltpu.async_copy(hbm_slc(k_local), acc_slc(acc), sem)

    def load_wait(k_local, acc, sem):
      pltpu.make_async_copy(hbm_slc(k_local), acc_slc(acc), sem).wait()

    def store_issue(k_local, acc, sem):
      pltpu.async_copy(acc_slc(acc), out_slc(k_local), sem)

    def store_wait(k_local, acc, sem):
      pltpu.make_async_copy(acc_slc(acc), out_slc(k_local), sem).wait()

    def work(k_local, acc):
      base = (k_local * NC + core) * R
      ones = lanes >= 0

      def scan_body(v, m_vec):
        vec = idxbuf[pl.ds(v * L, L)]
        rel = vec - base
        mask = rel.astype(jnp.uint32) < jnp.uint32(R)
        mi = mask.astype(jnp.int32)
        off = m_vec + plsc.cumsum(mi) - mi
        pos = lanes + (sub * SCAN_PER_TILE + v * L)
        plsc.store_scatter(posbuf, [off], pos, mask=mask)
        plsc.store_scatter(lidx2d, [off >> 7, off & (KB - 1)], rel, mask=mask)
        return m_vec + plsc.all_reduce_population_count(mask)

      m_vec = lax.fori_loop(0, NVEC, scan_body, jnp.zeros((L,), jnp.int32),
                            unroll=4)
      m = jnp.max(m_vec)

      @pl.loop(0, KB // L)
      def _pad(j):
        off_pad = m + lanes + j * L
        plsc.store_scatter(posbuf, [off_pad], lanes + j * L, mask=ones)
        plsc.store_scatter(lidx2d, [off_pad >> 7, off_pad & (KB - 1)],
                           lanes + R, mask=ones)

      nb = (m + (KB - 1)) // KB

      def batch_body(b, carry):
        pltpu.sync_copy(src_hbm.at[posbuf.at[pl.ds(b * KB, KB)]], staging)
        pltpu.sync_copy(staging, acc.at[lidx2d.at[b]], add=True)
        return carry

      lax.fori_loop(0, nb, batch_body, jnp.int32(0))

    pltpu.sync_copy(idx_hbm.at[pl.ds(sub * SCAN_PER_TILE, SCAN_PER_TILE)],
                    idxbuf)

    @pl.when(sub < DENSE_TILES)
    def _prime():
      load_issue(0, acc0, lsem0)
      load_issue(1, acc1, lsem1)

    @pl.loop(0, KPC // 2)
    def _pair(j):
      k0 = 2 * j
      k1 = 2 * j + 1

      @pl.when(sub < DENSE_TILES)
      def _w0():
        load_wait(k0, acc0, lsem0)
      plsc.subcore_barrier()
      work(k0, acc0)
      plsc.subcore_barrier()

      @pl.when(sub < DENSE_TILES)
      def _s0():
        store_issue(k0, acc0, ssem0)

      @pl.when(sub < DENSE_TILES)
      def _w1():
        load_wait(k1, acc1, lsem1)
      plsc.subcore_barrier()
      work(k1, acc1)
      plsc.subcore_barrier()

      @pl.when(sub < DENSE_TILES)
      def _s1():
        store_issue(k1, acc1, ssem1)

      @pl.when(jnp.logical_and(sub < DENSE_TILES, j < KPC // 2 - 1))
      def _reissue():
        store_wait(k0, acc0, ssem0)
        load_issue(k0 + 2, acc0, lsem0)
        store_wait(k1, acc1, ssem1)
        load_issue(k1 + 2, acc1, lsem1)

    @pl.when(sub < DENSE_TILES)
    def _drain():
      store_wait(KPC - 2, acc0, ssem0)
      store_wait(KPC - 1, acc1, ssem1)

  return run(dst, src, idx)


def kernel(dst_tensor, src_tensor, index_tensor):
  return _sc_index_add(dst_tensor, src_tensor,
                       index_tensor.astype(jnp.int32))
